# baseline (speedup 1.0000x reference)
import functools

import jax
import jax.numpy as jnp
from jax import lax
from jax.experimental import pallas as pl
from jax.experimental.pallas import tpu as pltpu
from jax.experimental.pallas import tpu_sc as plsc

_NUM_CORES = 2
_NUM_SUBCORES = 16
_NUM_WORKERS = _NUM_CORES * _NUM_SUBCORES

_SG = 2
_NROWS = 4
_NT = 2
_STAGE = 16


@functools.lru_cache(maxsize=None)
def _build(b: int, s: int, d: int):
    rw = b // _NUM_WORKERS
    n_grp = s // _SG
    assert rw * _NUM_WORKERS == b and rw == 128
    assert d == 64
    assert n_grp * _SG == s and n_grp % _NROWS == 0 and n_grp >= 3 * _NROWS

    mesh = plsc.VectorSubcoreMesh(core_axis_name="c", subcore_axis_name="s")

    @functools.partial(
        pl.kernel,
        out_type=jax.ShapeDtypeStruct((s, d // 8, b // 128, 8, 128),
                                      jnp.float32),
        mesh=mesh,
        scratch_types=[
            pltpu.VMEM((_STAGE, s), jnp.int32),
            pltpu.VMEM((s * rw,), jnp.int32),
            [pltpu.VMEM((_SG, rw, d), jnp.float32) for _ in range(_NROWS)],
            [pltpu.VMEM((_SG, 8, 8, 129), jnp.float32) for _ in range(_NT)],
            [pltpu.SemaphoreType.DMA for _ in range(_NROWS)],
            [pltpu.SemaphoreType.DMA for _ in range(_NT)],
        ],
        compiler_params=pltpu.CompilerParams(use_tc_tiling_on_sc=False,
                                             needs_layout_passes=False),
    )
    def gather_kernel(words_hbm, table_hbm, out5, stag, idx_t,
                      rows, ts, gsems, osems):
        wid = lax.axis_index("s") * _NUM_CORES + lax.axis_index("c")

        iota = lax.iota(jnp.int32, 16)
        zeros16 = jnp.zeros((16,), jnp.int32)
        dr0 = iota // 8
        di0 = iota % 8

        for k in range(rw // _STAGE):
            pltpu.sync_copy(
                words_hbm.at[pl.ds(wid * rw + k * _STAGE, _STAGE)], stag)

            def t_idx(s_, carry, k=k):
                sv = zeros16 + s_
                vs = [plsc.load_gather(stag, [iota + j * 16, sv])
                      for j in range(_STAGE // 16)]
                for j in range(_STAGE // 16):
                    idx_t[pl.ds(s_ * rw + k * _STAGE + j * 16, 16)] = vs[j]
                return carry

            lax.fori_loop(0, s, t_idx, 0)

        def g_start(g, r, sem):
            for j in range(_SG):
                pltpu.async_copy(
                    table_hbm.at[idx_t.at[pl.ds((g * _SG + j) * rw, rw)]],
                    r.at[j], sem)

        def g_wait(g, r, sem):
            for j in range(_SG):
                pltpu.make_async_copy(
                    table_hbm.at[idx_t.at[pl.ds((g * _SG + j) * rw, rw)]],
                    r.at[j], sem).wait()

        def s_start(g, t, sem):
            pltpu.async_copy(t.at[:, :, :, pl.ds(0, 128)],
                             out5.at[pl.ds(g * _SG, _SG), :, wid], sem)

        def s_wait(g, t, sem):
            pltpu.make_async_copy(
                t.at[:, :, :, pl.ds(0, 128)],
                out5.at[pl.ds(g * _SG, _SG), :, wid], sem).wait()

        def transpose(r, t):
            @plsc.parallel_loop(0, rw, unroll=4)
            def tb(bi):
                bv = zeros16 + bi
                vs = [r[sl, bi, pl.ds(d0, 16)]
                      for sl in range(_SG) for d0 in range(0, d, 16)]
                k = 0
                for sl in range(_SG):
                    for d0 in range(0, d, 16):
                        plsc.store_scatter(
                            t, [zeros16 + sl, dr0 + (d0 // 8), di0, bv],
                            vs[k])
                        k += 1

        n_blk = n_grp // _NROWS

        for u in range(_NROWS):
            g_start(u, rows[u], gsems[u])

        def blk(i, carry):
            for u in range(_NROWS):
                g = i * _NROWS + u
                tu = u % _NT

                if u >= _NT:
                    s_wait(g - _NT, ts[tu], osems[tu])
                else:
                    @pl.when(i > 0)
                    def _():
                        s_wait(g - _NT, ts[tu], osems[tu])

                g_wait(g, rows[u], gsems[u])
                transpose(rows[u], ts[tu])

                @pl.when(i < n_blk - 1)
                def _():
                    g_start(g + _NROWS, rows[u], gsems[u])

                s_start(g, ts[tu], osems[tu])
            return carry

        lax.fori_loop(0, n_blk, blk, 0)

        s_wait(n_grp - 2, ts[(_NROWS - 2) % _NT], osems[(_NROWS - 2) % _NT])
        s_wait(n_grp - 1, ts[(_NROWS - 1) % _NT], osems[(_NROWS - 1) % _NT])

    return gather_kernel


def kernel(words, table):
    b, s = words.shape
    _, d = table.shape
    out5 = _build(b, s, d)(words.astype(jnp.int32), table)
    return jnp.transpose(out5, (2, 4, 0, 1, 3)).reshape(b, s, d)
